# baseline (speedup 1.0000x reference)
import functools

import jax
import jax.numpy as jnp
from jax import lax
from jax.experimental import pallas as pl
from jax.experimental.pallas import tpu as pltpu
from jax.experimental.pallas import tpu_sc as plsc

D = 256
NP = 10240
EB = 128
NEG = -3.0e38


def _k1_body(w1t_ref, w2t_ref, w3t_ref, brow_ref, wct_ref, cvec_ref):
    t1 = jnp.dot(w2t_ref[:], w3t_ref[:], preferred_element_type=jnp.float32)
    wct_ref[:] = jnp.dot(w1t_ref[:], t1, preferred_element_type=jnp.float32)
    b1 = brow_ref[0:1, :]
    bc = jnp.dot(b1, t1, preferred_element_type=jnp.float32)
    bc = bc + jnp.dot(brow_ref[1:2, 0:512], w3t_ref[:],
                      preferred_element_type=jnp.float32)
    bc = bc + brow_ref[2:3, 0:D]
    p = bc + brow_ref[3:4, 0:D]
    w = brow_ref[4:5, 0:D] * brow_ref[6:7, 0:D]
    pw = jnp.sum(p * w)
    mp = jnp.mean(p)
    mpp = jnp.mean(p * p)
    sumw = jnp.sum(w)
    c0 = jnp.sum(brow_ref[5:6, 0:D] * brow_ref[6:7, 0:D]) + brow_ref[7, 0]
    sub = lax.broadcasted_iota(jnp.int32, (8, D), 0)
    cv = jnp.where(sub == 0, jnp.broadcast_to(bc, (8, D)), 0.0)
    cv = jnp.where(sub == 1, jnp.broadcast_to(p, (8, D)), cv)
    cv = jnp.where(sub == 2, jnp.broadcast_to(w, (8, D)), cv)
    cv = jnp.where(sub == 3, pw, cv)
    cv = jnp.where(sub == 4, mp, cv)
    cv = jnp.where(sub == 5, mpp, cv)
    cv = jnp.where(sub == 6, sumw, cv)
    cv = jnp.where(sub == 7, c0, cv)
    cvec_ref[:] = cv


def _fold_weights(w1t, w2t, w3t, brow):
    return pl.pallas_call(
        _k1_body,
        out_shape=(jax.ShapeDtypeStruct((D, D), jnp.float32),
                   jax.ShapeDtypeStruct((8, D), jnp.float32)),
    )(w1t, w2t, w3t, brow)


_NB = 512


def _k2_body(x_ref, wqt_ref, wkt_ref, wvt_ref, wct_ref, vrows_ref, cvec_ref,
             q_ref, k_ref, s_ref):
    xb = x_ref[:]
    q = jnp.dot(xb, wqt_ref[:], preferred_element_type=jnp.float32)
    q = q + vrows_ref[0:1, :]
    k = jnp.dot(xb, wkt_ref[:], preferred_element_type=jnp.float32)
    k = k + vrows_ref[1:2, :]
    v = jnp.dot(xb, wvt_ref[:], preferred_element_type=jnp.float32)
    v = v + vrows_ref[2:3, :]
    rv = jnp.maximum(v, 0.0)
    u = jnp.dot(rv, wct_ref[:], preferred_element_type=jnp.float32)
    q_ref[:] = q
    k_ref[:] = k
    g = vrows_ref[3:4, :]
    p = cvec_ref[1:2, :]
    w = cvec_ref[2:3, :]
    mx = jnp.mean(xb, axis=1, keepdims=True)
    xc = xb - mx
    mr = jnp.mean(rv, axis=1, keepdims=True)
    rc = rv - mr
    xg = xc * g
    rg = rc * g
    cols = [
        jnp.mean(xc * xc, axis=1), jnp.mean(rc * rc, axis=1),
        jnp.mean(xc * rc, axis=1),
        jnp.sum(u * w, axis=1), jnp.sum(xg * w, axis=1),
        jnp.sum(rg * w, axis=1),
        jnp.mean(u, axis=1), jnp.mean(xg, axis=1), jnp.mean(rg, axis=1),
        jnp.mean(u * u, axis=1), jnp.mean(u * p, axis=1),
        jnp.mean(u * xg, axis=1), jnp.mean(u * rg, axis=1),
        jnp.mean(p * xg, axis=1), jnp.mean(p * rg, axis=1),
        jnp.mean(xg * xg, axis=1), jnp.mean(xg * rg, axis=1),
        jnp.mean(rg * rg, axis=1),
    ]
    ii = lax.broadcasted_iota(jnp.int32, (_NB, 128), 1)
    acc = jnp.zeros((_NB, 128), jnp.float32)
    for j, c in enumerate(cols):
        acc = jnp.where(ii == j, c[:, None], acc)
    s_ref[:] = acc


def _node_precompute(xpad, wqt, wkt, wvt, wct, vrows, cvec):
    nblk = NP // _NB
    full = lambda shape: pl.BlockSpec(shape, lambda i: (0, 0))
    return pl.pallas_call(
        _k2_body,
        grid=(nblk,),
        in_specs=[
            pl.BlockSpec((_NB, D), lambda i: (i, 0)),
            full((D, D)), full((D, D)), full((D, D)), full((D, D)),
            full((8, D)), full((8, D)),
        ],
        out_specs=[
            pl.BlockSpec((_NB, D), lambda i: (i, 0)),
            pl.BlockSpec((_NB, D), lambda i: (i, 0)),
            pl.BlockSpec((_NB, 128), lambda i: (i, 0)),
        ],
        out_shape=(jax.ShapeDtypeStruct((NP, D), jnp.float32),
                   jax.ShapeDtypeStruct((NP, D), jnp.float32),
                   jax.ShapeDtypeStruct((NP, 128), jnp.float32)),
    )(xpad, wqt, wkt, wvt, wct, vrows, cvec)


_MESH = plsc.VectorSubcoreMesh(core_axis_name="c", subcore_axis_name="s")
_NC = 2
_NS = 16
_NW = _NC * _NS
_SCPARAMS = pltpu.CompilerParams(use_tc_tiling_on_sc=False,
                                 needs_layout_passes=False)

EB3 = 64
EBF = 128
EP = 163840
_R3 = EP // EB3 // _NW
_RF = EP // EBF // _NW


def _worker_id():
    return lax.axis_index("s") * _NC + lax.axis_index("c")


_GDN = lax.GatherDimensionNumbers(offset_dims=(), collapsed_slice_dims=(0,),
                                  start_index_map=(0,))


def _lanegather(a, i):
    return lax.gather(a, i[:, None], _GDN, slice_sizes=(1,),
                      mode=lax.GatherScatterMode.PROMISE_IN_BOUNDS)


def _rsqrt16(x):
    i = plsc.bitcast(x, jnp.int32)
    i = 0x5F3759DF - lax.shift_right_logical(i, 1)
    y = plsc.bitcast(i, jnp.float32)
    for _ in range(4):
        y = y * (1.5 - 0.5 * x * y * y)
    return y


def _fill(ref, n16, val):
    def z(t, carry):
        ref[pl.ds(t * 16, 16)] = jnp.full((16,), val, jnp.float32)
        return carry
    lax.fori_loop(0, n16, z, 0)


def _k3_body(src_hbm, dst_hbm, q_hbm, k_hbm, s128_hbm,
             araw_hbm, mpart_hbm, s32_hbm,
             srcA, dstA, srcB, dstB, qA, kA, qB, kB, araw_v, mpriv,
             buf_a, buf_b, schunk, s32buf, mshare, semA, semB):
    w = _worker_id()
    cid = lax.axis_index("c")
    sid = lax.axis_index("s")

    nbase = w * (NP // _NW)
    for c in range(NP // _NW // EB3):
        pltpu.sync_copy(s128_hbm.at[pl.ds(nbase + c * EB3, EB3)], schunk)

        def prow(i, carry):
            s32buf[i, pl.ds(0, 16)] = schunk[i, pl.ds(0, 16)]
            s32buf[i, pl.ds(16, 16)] = schunk[i, pl.ds(16, 16)]
            return carry
        lax.fori_loop(0, EB3, prow, 0)
        pltpu.sync_copy(s32buf, s32_hbm.at[pl.ds(nbase + c * EB3, EB3)])

    _fill(mpriv, NP // 16, NEG)

    rbase = w * _R3

    def fire(r, src_v, dst_v, q_v, k_v, sem):
        base = (rbase + r) * EB3
        pltpu.sync_copy(src_hbm.at[pl.ds(base, EB3)], src_v)
        pltpu.sync_copy(dst_hbm.at[pl.ds(base, EB3)], dst_v)
        pltpu.async_copy(q_hbm.at[src_v], q_v, sem)
        pltpu.async_copy(k_hbm.at[dst_v], k_v, sem)

    def drain(q_v, k_v, sem):
        pltpu.make_async_copy(q_hbm.at[pl.ds(0, EB3)], q_v, sem).wait()
        pltpu.make_async_copy(k_hbm.at[pl.ds(0, EB3)], k_v, sem).wait()

    def compute(r, dst_v, q_v, k_v):
        lane = lax.iota(jnp.int32, 16)
        shuf = [jnp.bitwise_xor(lane, sh) for sh in (8, 4, 2, 1)]
        for t in range(EB3 // 16):

            def edge(i, out_vec):
                e = t * 16 + i
                acc = jnp.zeros((16,), jnp.float32)
                for c in range(2):
                    qs = [q_v[e, pl.ds((c * 8 + u) * 16, 16)]
                          for u in range(8)]
                    ks = [k_v[e, pl.ds((c * 8 + u) * 16, 16)]
                          for u in range(8)]
                    pr = [qs[u] * ks[u] for u in range(8)]
                    acc = acc + (((pr[0] + pr[1]) + (pr[2] + pr[3]))
                                 + ((pr[4] + pr[5]) + (pr[6] + pr[7])))
                for sx in shuf:
                    acc = acc + _lanegather(acc, sx)
                return jnp.where(lane == i, acc, out_vec)

            out_vec = lax.fori_loop(0, 16, edge,
                                    jnp.zeros((16,), jnp.float32))
            araw_v[pl.ds(t * 16, 16)] = out_vec
            didx = dst_v[pl.ds(t * 16, 16)]
            cur = plsc.load_gather(mpriv, [didx])
            plsc.store_scatter(mpriv, [didx], jnp.maximum(cur, out_vec))
        pltpu.sync_copy(araw_v, araw_hbm.at[pl.ds((rbase + r) * EB3, EB3)])

    fire(0, srcA, dstA, qA, kA, semA)

    def pair(i, carry):
        g = i * 2
        fire(g + 1, srcB, dstB, qB, kB, semB)
        drain(qA, kA, semA)
        compute(g, dstA, qA, kA)

        @pl.when(g + 2 < _R3)
        def _():
            fire(g + 2, srcA, dstA, qA, kA, semA)
        drain(qB, kB, semB)
        compute(g + 1, dstB, qB, kB)
        return carry

    lax.fori_loop(0, _R3 // 2, pair, 0)

    pltpu.sync_copy(mpriv, mshare.at[sid])
    plsc.subcore_barrier()
    sl = NP // _NS
    pltpu.sync_copy(mshare.at[0, pl.ds(sid * sl, sl)], buf_a)
    for j in range(1, _NS):
        pltpu.sync_copy(mshare.at[j, pl.ds(sid * sl, sl)], buf_b)

        def mx(t, carry):
            buf_a[pl.ds(t * 16, 16)] = jnp.maximum(buf_a[pl.ds(t * 16, 16)],
                                                   buf_b[pl.ds(t * 16, 16)])
            return carry
        lax.fori_loop(0, sl // 16, mx, 0)
    pltpu.sync_copy(buf_a, mpart_hbm.at[pl.ds(cid * NP + sid * sl, sl)])


def _edge_dots(src, dst, q, k, s128):
    kern = pl.kernel(
        _k3_body,
        out_type=(jax.ShapeDtypeStruct((EP,), jnp.float32),
                  jax.ShapeDtypeStruct((_NC * NP,), jnp.float32),
                  jax.ShapeDtypeStruct((NP, 32), jnp.float32)),
        mesh=_MESH,
        compiler_params=_SCPARAMS,
        scratch_types=[
            pltpu.VMEM((EB3,), jnp.int32),
            pltpu.VMEM((EB3,), jnp.int32),
            pltpu.VMEM((EB3,), jnp.int32),
            pltpu.VMEM((EB3,), jnp.int32),
            pltpu.VMEM((EB3, D), jnp.float32),
            pltpu.VMEM((EB3, D), jnp.float32),
            pltpu.VMEM((EB3, D), jnp.float32),
            pltpu.VMEM((EB3, D), jnp.float32),
            pltpu.VMEM((EB3,), jnp.float32),
            pltpu.VMEM((NP,), jnp.float32),
            pltpu.VMEM((NP // _NS,), jnp.float32),
            pltpu.VMEM((NP // _NS,), jnp.float32),
            pltpu.VMEM((EB3, 128), jnp.float32),
            pltpu.VMEM((EB3, 32), jnp.float32),
            pltpu.VMEM_SHARED((_NS, NP), jnp.float32),
            pltpu.SemaphoreType.DMA,
            pltpu.SemaphoreType.DMA,
        ],
    )
    return kern(src, dst, q, k, s128)


def _k4_body(araw_hbm, dst_hbm, mpart_hbm, e_hbm, spart_hbm,
             a_v, dst_v, e_v, mloc, buf, sshare, sem):
    w = _worker_id()
    cid = lax.axis_index("c")
    sid = lax.axis_index("s")
    pltpu.sync_copy(mpart_hbm.at[pl.ds(0, NP)], mloc)
    pltpu.sync_copy(mpart_hbm.at[pl.ds(NP, NP)], buf)

    def mx(t, carry):
        mloc[pl.ds(t * 16, 16)] = jnp.maximum(mloc[pl.ds(t * 16, 16)],
                                              buf[pl.ds(t * 16, 16)])
        return carry
    lax.fori_loop(0, NP // 16, mx, 0)

    _fill(buf, (NP // _NS) // 16, 0.0)
    pltpu.sync_copy(buf.at[pl.ds(0, NP // _NS)],
                    sshare.at[pl.ds(sid * (NP // _NS), NP // _NS)])
    plsc.subcore_barrier()

    def row(i, carry):
        base = (w * _RF + i) * EBF
        pltpu.sync_copy(araw_hbm.at[pl.ds(base, EBF)], a_v)
        pltpu.sync_copy(dst_hbm.at[pl.ds(base, EBF)], dst_v)
        for t in range(EBF // 16):
            didx = dst_v[pl.ds(t * 16, 16)]
            mg = plsc.load_gather(mloc, [didx])
            e_v[pl.ds(t * 16, 16)] = jnp.exp(a_v[pl.ds(t * 16, 16)] - mg)
        pltpu.sync_copy(e_v, e_hbm.at[pl.ds(base, EBF)])
        pltpu.sync_copy(e_v, sshare.at[dst_v], add=True)
        return carry

    lax.fori_loop(0, _RF, row, 0)
    plsc.subcore_barrier()

    @pl.when(sid == 0)
    def _():
        pltpu.sync_copy(sshare, buf)
        pltpu.sync_copy(buf, spart_hbm.at[pl.ds(cid * NP, NP)])


def _edge_exp_sums(araw, dst, mpart):
    kern = pl.kernel(
        _k4_body,
        out_type=(jax.ShapeDtypeStruct((EP,), jnp.float32),
                  jax.ShapeDtypeStruct((_NC * NP,), jnp.float32)),
        mesh=_MESH,
        compiler_params=_SCPARAMS,
        scratch_types=[
            pltpu.VMEM((EBF,), jnp.float32),
            pltpu.VMEM((EBF,), jnp.int32),
            pltpu.VMEM((EBF,), jnp.float32),
            pltpu.VMEM((NP,), jnp.float32),
            pltpu.VMEM((NP,), jnp.float32),
            pltpu.VMEM_SHARED((NP,), jnp.float32),
            pltpu.SemaphoreType.DMA,
        ],
    )
    return kern(araw, dst, mpart)


def _k5_body(e_hbm, dst_hbm, spart_hbm, stab_hbm, cvec_hbm, out_hbm,
             e_vA, dst_vA, srowsA, e_vB, dst_vB, srowsB, out_v,
             sloc, buf, cvec_v, s35, semA, semB):
    w = _worker_id()
    pltpu.sync_copy(spart_hbm.at[pl.ds(0, NP)], sloc)
    pltpu.sync_copy(spart_hbm.at[pl.ds(NP, NP)], buf)

    def ad(t, carry):
        sloc[pl.ds(t * 16, 16)] = (sloc[pl.ds(t * 16, 16)]
                                   + buf[pl.ds(t * 16, 16)])
        return carry
    lax.fori_loop(0, NP // 16, ad, 0)
    pltpu.sync_copy(cvec_hbm, cvec_v)
    pwv = cvec_v[3, pl.ds(0, 16)]
    mpv = cvec_v[4, pl.ds(0, 16)]
    mppv = cvec_v[5, pl.ds(0, 16)]
    sumwv = cvec_v[6, pl.ds(0, 16)]
    c0v = cvec_v[7, pl.ds(0, 16)]

    def fire(i, e_v, dst_v, srows, sem):
        base = (w * _RF + i) * EBF
        pltpu.sync_copy(e_hbm.at[pl.ds(base, EBF)], e_v)
        pltpu.sync_copy(dst_hbm.at[pl.ds(base, EBF)], dst_v)
        pltpu.async_copy(stab_hbm.at[dst_v], srows, sem)

    def drain(srows, sem):
        pltpu.make_async_copy(stab_hbm.at[pl.ds(0, EBF)], srows, sem).wait()

    def compute(i, e_v, dst_v, srows):
        base = (w * _RF + i) * EBF

        def restride(n, carry):
            s35[n, pl.ds(0, 16)] = srows[n, pl.ds(0, 16)]
            s35[n, pl.ds(16, 16)] = srows[n, pl.ds(16, 16)]
            return carry
        lax.fori_loop(0, EBF, restride, 0)
        for t in range(EBF // 16):
            didx = dst_v[pl.ds(t * 16, 16)]
            sg = plsc.load_gather(sloc, [didx])
            a = e_v[pl.ds(t * 16, 16)] / (sg + 1e-16)
            ridx = t * 16 + lax.iota(jnp.int32, 16)

            cols = [plsc.load_gather(
                        s35, [ridx, jnp.full((16,), j, jnp.int32)])
                    for j in range(18)]
            (vx, vr, cxr, uw, xgw, rgw, mu, mxg, mrg, muu, mup,
             m_uxg, m_urg, m_pxg, m_prg, m_xg2, m_xgrg, m_rg2) = cols
            a2 = a * a
            s1sq = vx + 2.0 * a * cxr + a2 * vr + 1e-5
            rs1 = _rsqrt16(s1sq)
            zw = a * uw + pwv + (xgw + a * rgw) * rs1
            muz = a * mu + mpv + (mxg + a * mrg) * rs1
            m_a2 = a2 * muu + 2.0 * a * mup + mppv
            m_ab = a * m_uxg + a2 * m_urg + m_pxg + a * m_prg
            m_b2 = m_xg2 + 2.0 * a * m_xgrg + a2 * m_rg2
            varz = m_a2 + 2.0 * m_ab * rs1 + m_b2 / s1sq - muz * muz
            out_v[pl.ds(t * 16, 16)] = ((zw - muz * sumwv)
                                        * _rsqrt16(varz + 1e-5) + c0v)
        pltpu.sync_copy(out_v, out_hbm.at[pl.ds(base, EBF)])

    fire(0, e_vA, dst_vA, srowsA, semA)

    def pair(i, carry):
        g = i * 2
        fire(g + 1, e_vB, dst_vB, srowsB, semB)
        drain(srowsA, semA)
        compute(g, e_vA, dst_vA, srowsA)

        @pl.when(g + 2 < _RF)
        def _():
            fire(g + 2, e_vA, dst_vA, srowsA, semA)
        drain(srowsB, semB)
        compute(g + 1, e_vB, dst_vB, srowsB)
        return carry

    lax.fori_loop(0, _RF // 2, pair, 0)


def _edge_final(ev, dst, spart, stab, cvec):
    kern = pl.kernel(
        _k5_body,
        out_type=jax.ShapeDtypeStruct((EP,), jnp.float32),
        mesh=_MESH,
        compiler_params=_SCPARAMS,
        scratch_types=[
            pltpu.VMEM((EBF,), jnp.float32),
            pltpu.VMEM((EBF,), jnp.int32),
            pltpu.VMEM((EBF, 32), jnp.float32),
            pltpu.VMEM((EBF,), jnp.float32),
            pltpu.VMEM((EBF,), jnp.int32),
            pltpu.VMEM((EBF, 32), jnp.float32),
            pltpu.VMEM((EBF,), jnp.float32),
            pltpu.VMEM((NP,), jnp.float32),
            pltpu.VMEM((NP,), jnp.float32),
            pltpu.VMEM((8, D), jnp.float32),
            pltpu.VMEM((EBF, 35), jnp.float32),
            pltpu.SemaphoreType.DMA,
            pltpu.SemaphoreType.DMA,
        ],
    )
    return kern(ev, dst, spart, stab, cvec)


def kernel(edge_index, x, Wq, bq, Wk, bk, Wv, bv, ln_g, ln_b,
           W1, b1, W2, b2, W3, b3, Wvec, bvec, fn_g, fn_b):
    ei = edge_index.astype(jnp.int32)
    ne = ei.shape[1]
    src = jnp.pad(ei[0], (0, EP - ne))
    dst = jnp.pad(ei[1], (0, EP - ne), constant_values=NP - 1)
    L = Wq.shape[0] - 1

    brow = jnp.zeros((8, 1024), jnp.float32)
    brow = brow.at[0, :].set(b1)
    brow = brow.at[1, :512].set(b2)
    brow = brow.at[2, :D].set(b3)
    brow = brow.at[3, :D].set(ln_b[L])
    brow = brow.at[4, :D].set(fn_g)
    brow = brow.at[5, :D].set(fn_b)
    brow = brow.at[6, :D].set(Wvec[0])
    brow = brow.at[7, :].set(bvec[0])
    wct, cvec = _fold_weights(W1.T, W2.T, W3.T, brow)

    xpad = jnp.pad(x, ((0, NP - x.shape[0]), (0, 0)))
    vrows = jnp.zeros((8, D), jnp.float32)
    vrows = vrows.at[0].set(bq[L]).at[1].set(bk[L]).at[2].set(bv[L])
    vrows = vrows.at[3].set(ln_g[L])
    q, k, s128 = _node_precompute(xpad, Wq[L].T, Wk[L].T, Wv[L].T, wct,
                                  vrows, cvec)

    araw, mpart, s32 = _edge_dots(src, dst, q, k, s128)
    ev, spart = _edge_exp_sums(araw, dst, mpart)
    return _edge_final(ev, dst, spart, s32, cvec)[:ne]


# speedup vs baseline: 1.0096x; 1.0096x over previous
import jax
import jax.numpy as jnp
from jax import lax
from jax.experimental import pallas as pl
from jax.experimental.pallas import tpu as pltpu
from jax.experimental.pallas import tpu_sc as plsc

D = 256
NP = 10240
EB = 128
NEG = -3.0e38


def _k1_body(w1t_ref, w2t_ref, w3t_ref, brow_ref, wct_ref, cvec_ref):
    t1 = jnp.dot(w2t_ref[:], w3t_ref[:], preferred_element_type=jnp.float32)
    wct_ref[:] = jnp.dot(w1t_ref[:], t1, preferred_element_type=jnp.float32)
    b1 = brow_ref[0:1, :]
    bc = jnp.dot(b1, t1, preferred_element_type=jnp.float32)
    bc = bc + jnp.dot(brow_ref[1:2, 0:512], w3t_ref[:],
                      preferred_element_type=jnp.float32)
    bc = bc + brow_ref[2:3, 0:D]
    p = bc + brow_ref[3:4, 0:D]
    w = brow_ref[4:5, 0:D] * brow_ref[6:7, 0:D]
    pw = jnp.sum(p * w)
    mp = jnp.mean(p)
    mpp = jnp.mean(p * p)
    sumw = jnp.sum(w)
    c0 = jnp.sum(brow_ref[5:6, 0:D] * brow_ref[6:7, 0:D]) + brow_ref[7, 0]
    sub = lax.broadcasted_iota(jnp.int32, (8, D), 0)
    cv = jnp.where(sub == 0, jnp.broadcast_to(bc, (8, D)), 0.0)
    cv = jnp.where(sub == 1, jnp.broadcast_to(p, (8, D)), cv)
    cv = jnp.where(sub == 2, jnp.broadcast_to(w, (8, D)), cv)
    cv = jnp.where(sub == 3, pw, cv)
    cv = jnp.where(sub == 4, mp, cv)
    cv = jnp.where(sub == 5, mpp, cv)
    cv = jnp.where(sub == 6, sumw, cv)
    cv = jnp.where(sub == 7, c0, cv)
    cvec_ref[:] = cv


def _fold_weights(w1t, w2t, w3t, brow):
    return pl.pallas_call(
        _k1_body,
        out_shape=(jax.ShapeDtypeStruct((D, D), jnp.float32),
                   jax.ShapeDtypeStruct((8, D), jnp.float32)),
    )(w1t, w2t, w3t, brow)


_NB = 512


def _k2_body(x_ref, wqt_ref, wkt_ref, wvt_ref, wct_ref, vrows_ref, cvec_ref,
             q_ref, k_ref, s_ref):
    xb = x_ref[:]
    q = jnp.dot(xb, wqt_ref[:], preferred_element_type=jnp.float32)
    q = q + vrows_ref[0:1, :]
    k = jnp.dot(xb, wkt_ref[:], preferred_element_type=jnp.float32)
    k = k + vrows_ref[1:2, :]
    v = jnp.dot(xb, wvt_ref[:], preferred_element_type=jnp.float32)
    v = v + vrows_ref[2:3, :]
    rv = jnp.maximum(v, 0.0)
    u = jnp.dot(rv, wct_ref[:], preferred_element_type=jnp.float32)
    q_ref[:] = q
    k_ref[:] = k
    g = vrows_ref[3:4, :]
    p = cvec_ref[1:2, :]
    w = cvec_ref[2:3, :]
    mx = jnp.mean(xb, axis=1, keepdims=True)
    xc = xb - mx
    mr = jnp.mean(rv, axis=1, keepdims=True)
    rc = rv - mr
    xg = xc * g
    rg = rc * g
    cols = [
        jnp.mean(xc * xc, axis=1), jnp.mean(rc * rc, axis=1),
        jnp.mean(xc * rc, axis=1),
        jnp.sum(u * w, axis=1), jnp.sum(xg * w, axis=1),
        jnp.sum(rg * w, axis=1),
        jnp.mean(u, axis=1), jnp.mean(xg, axis=1), jnp.mean(rg, axis=1),
        jnp.mean(u * u, axis=1), jnp.mean(u * p, axis=1),
        jnp.mean(u * xg, axis=1), jnp.mean(u * rg, axis=1),
        jnp.mean(p * xg, axis=1), jnp.mean(p * rg, axis=1),
        jnp.mean(xg * xg, axis=1), jnp.mean(xg * rg, axis=1),
        jnp.mean(rg * rg, axis=1),
    ]
    ii = lax.broadcasted_iota(jnp.int32, (_NB, 128), 1)
    acc = jnp.zeros((_NB, 128), jnp.float32)
    for j, c in enumerate(cols):
        acc = jnp.where(ii == j, c[:, None], acc)
    s_ref[:] = acc


def _node_precompute(xpad, wqt, wkt, wvt, wct, vrows, cvec):
    nblk = NP // _NB
    full = lambda shape: pl.BlockSpec(shape, lambda i: (0, 0))
    return pl.pallas_call(
        _k2_body,
        grid=(nblk,),
        in_specs=[
            pl.BlockSpec((_NB, D), lambda i: (i, 0)),
            full((D, D)), full((D, D)), full((D, D)), full((D, D)),
            full((8, D)), full((8, D)),
        ],
        out_specs=[
            pl.BlockSpec((_NB, D), lambda i: (i, 0)),
            pl.BlockSpec((_NB, D), lambda i: (i, 0)),
            pl.BlockSpec((_NB, 128), lambda i: (i, 0)),
        ],
        out_shape=(jax.ShapeDtypeStruct((NP, D), jnp.float32),
                   jax.ShapeDtypeStruct((NP, D), jnp.float32),
                   jax.ShapeDtypeStruct((NP, 128), jnp.float32)),
    )(xpad, wqt, wkt, wvt, wct, vrows, cvec)


_MESH = plsc.VectorSubcoreMesh(core_axis_name="c", subcore_axis_name="s")
_NC = 2
_NS = 16
_NW = _NC * _NS
_SCPARAMS = pltpu.CompilerParams(use_tc_tiling_on_sc=False,
                                 needs_layout_passes=False)

EB3 = 64
EBF = 128
EP = 163840
_R3 = EP // EB3 // _NW
_RF = EP // EBF // _NW


def _worker_id():
    return lax.axis_index("s") * _NC + lax.axis_index("c")


_GDN = lax.GatherDimensionNumbers(offset_dims=(), collapsed_slice_dims=(0,),
                                  start_index_map=(0,))


def _lanegather(a, i):
    return lax.gather(a, i[:, None], _GDN, slice_sizes=(1,),
                      mode=lax.GatherScatterMode.PROMISE_IN_BOUNDS)


def _rsqrt16(x):
    i = plsc.bitcast(x, jnp.int32)
    i = 0x5F3759DF - lax.shift_right_logical(i, 1)
    y = plsc.bitcast(i, jnp.float32)
    for _ in range(4):
        y = y * (1.5 - 0.5 * x * y * y)
    return y


def _fill(ref, n16, val):
    def z(t, carry):
        ref[pl.ds(t * 16, 16)] = jnp.full((16,), val, jnp.float32)
        return carry
    lax.fori_loop(0, n16, z, 0)


def _k3_body(src_hbm, dst_hbm, q_hbm, k_hbm, s128_hbm,
             araw_hbm, mpart_hbm, s32_hbm,
             srcA, dstA, srcB, dstB, qA, kA, qB, kB, araw_v, mpriv,
             buf_a, buf_b, schunk, s32buf, mshare, semA, semB):
    w = _worker_id()
    cid = lax.axis_index("c")
    sid = lax.axis_index("s")

    nbase = w * (NP // _NW)
    for c in range(NP // _NW // EB3):
        pltpu.sync_copy(s128_hbm.at[pl.ds(nbase + c * EB3, EB3)], schunk)

        def prow(i, carry):
            s32buf[i, pl.ds(0, 16)] = schunk[i, pl.ds(0, 16)]
            s32buf[i, pl.ds(16, 16)] = schunk[i, pl.ds(16, 16)]
            return carry
        lax.fori_loop(0, EB3, prow, 0)
        pltpu.sync_copy(s32buf, s32_hbm.at[pl.ds(nbase + c * EB3, EB3)])

    _fill(mpriv, NP // 16, NEG)

    rbase = w * _R3

    def fire(r, src_v, dst_v, q_v, k_v, sem):
        base = (rbase + r) * EB3
        pltpu.sync_copy(src_hbm.at[pl.ds(base, EB3)], src_v)
        pltpu.sync_copy(dst_hbm.at[pl.ds(base, EB3)], dst_v)
        pltpu.async_copy(q_hbm.at[src_v], q_v, sem)
        pltpu.async_copy(k_hbm.at[dst_v], k_v, sem)

    def drain(q_v, k_v, sem):
        pltpu.make_async_copy(q_hbm.at[pl.ds(0, EB3)], q_v, sem).wait()
        pltpu.make_async_copy(k_hbm.at[pl.ds(0, EB3)], k_v, sem).wait()

    def compute(r, dst_v, q_v, k_v):
        lane = lax.iota(jnp.int32, 16)
        shuf = [jnp.bitwise_xor(lane, sh) for sh in (8, 4, 2, 1)]
        for t in range(EB3 // 16):

            def edge(i, out_vec):
                e = t * 16 + i
                acc = jnp.zeros((16,), jnp.float32)
                for c in range(2):
                    qs = [q_v[e, pl.ds((c * 8 + u) * 16, 16)]
                          for u in range(8)]
                    ks = [k_v[e, pl.ds((c * 8 + u) * 16, 16)]
                          for u in range(8)]
                    pr = [qs[u] * ks[u] for u in range(8)]
                    acc = acc + (((pr[0] + pr[1]) + (pr[2] + pr[3]))
                                 + ((pr[4] + pr[5]) + (pr[6] + pr[7])))
                for sx in shuf:
                    acc = acc + _lanegather(acc, sx)
                return jnp.where(lane == i, acc, out_vec)

            out_vec = lax.fori_loop(0, 16, edge,
                                    jnp.zeros((16,), jnp.float32))
            araw_v[pl.ds(t * 16, 16)] = out_vec
            didx = dst_v[pl.ds(t * 16, 16)]
            cur = plsc.load_gather(mpriv, [didx])
            plsc.store_scatter(mpriv, [didx], jnp.maximum(cur, out_vec))
        pltpu.sync_copy(araw_v, araw_hbm.at[pl.ds((rbase + r) * EB3, EB3)])

    fire(0, srcA, dstA, qA, kA, semA)

    def pair(i, carry):
        g = i * 2
        fire(g + 1, srcB, dstB, qB, kB, semB)
        drain(qA, kA, semA)
        compute(g, dstA, qA, kA)

        @pl.when(g + 2 < _R3)
        def _():
            fire(g + 2, srcA, dstA, qA, kA, semA)
        drain(qB, kB, semB)
        compute(g + 1, dstB, qB, kB)
        return carry

    lax.fori_loop(0, _R3 // 2, pair, 0)

    pltpu.sync_copy(mpriv, mshare.at[sid])
    plsc.subcore_barrier()
    sl = NP // _NS
    pltpu.sync_copy(mshare.at[0, pl.ds(sid * sl, sl)], buf_a)
    for j in range(1, _NS):
        pltpu.sync_copy(mshare.at[j, pl.ds(sid * sl, sl)], buf_b)

        def mx(t, carry):
            buf_a[pl.ds(t * 16, 16)] = jnp.maximum(buf_a[pl.ds(t * 16, 16)],
                                                   buf_b[pl.ds(t * 16, 16)])
            return carry
        lax.fori_loop(0, sl // 16, mx, 0)
    pltpu.sync_copy(buf_a, mpart_hbm.at[pl.ds(cid * NP + sid * sl, sl)])


def _edge_dots(src, dst, q, k, s128):
    kern = pl.kernel(
        _k3_body,
        out_type=(jax.ShapeDtypeStruct((EP,), jnp.float32),
                  jax.ShapeDtypeStruct((_NC * NP,), jnp.float32),
                  jax.ShapeDtypeStruct((NP, 32), jnp.float32)),
        mesh=_MESH,
        compiler_params=_SCPARAMS,
        scratch_types=[
            pltpu.VMEM((EB3,), jnp.int32),
            pltpu.VMEM((EB3,), jnp.int32),
            pltpu.VMEM((EB3,), jnp.int32),
            pltpu.VMEM((EB3,), jnp.int32),
            pltpu.VMEM((EB3, D), jnp.float32),
            pltpu.VMEM((EB3, D), jnp.float32),
            pltpu.VMEM((EB3, D), jnp.float32),
            pltpu.VMEM((EB3, D), jnp.float32),
            pltpu.VMEM((EB3,), jnp.float32),
            pltpu.VMEM((NP,), jnp.float32),
            pltpu.VMEM((NP // _NS,), jnp.float32),
            pltpu.VMEM((NP // _NS,), jnp.float32),
            pltpu.VMEM((EB3, 128), jnp.float32),
            pltpu.VMEM((EB3, 32), jnp.float32),
            pltpu.VMEM_SHARED((_NS, NP), jnp.float32),
            pltpu.SemaphoreType.DMA,
            pltpu.SemaphoreType.DMA,
        ],
    )
    return kern(src, dst, q, k, s128)


def _k4_body(araw_hbm, dst_hbm, mpart_hbm, e_hbm, spart_hbm,
             a_v, dst_v, e_v, mloc, buf, sshare, sem):
    w = _worker_id()
    cid = lax.axis_index("c")
    sid = lax.axis_index("s")
    pltpu.sync_copy(mpart_hbm.at[pl.ds(0, NP)], mloc)
    pltpu.sync_copy(mpart_hbm.at[pl.ds(NP, NP)], buf)

    def mx(t, carry):
        mloc[pl.ds(t * 16, 16)] = jnp.maximum(mloc[pl.ds(t * 16, 16)],
                                              buf[pl.ds(t * 16, 16)])
        return carry
    lax.fori_loop(0, NP // 16, mx, 0)

    _fill(buf, (NP // _NS) // 16, 0.0)
    pltpu.sync_copy(buf.at[pl.ds(0, NP // _NS)],
                    sshare.at[pl.ds(sid * (NP // _NS), NP // _NS)])
    plsc.subcore_barrier()

    def row(i, carry):
        base = (w * _RF + i) * EBF
        pltpu.sync_copy(araw_hbm.at[pl.ds(base, EBF)], a_v)
        pltpu.sync_copy(dst_hbm.at[pl.ds(base, EBF)], dst_v)
        for t in range(EBF // 16):
            didx = dst_v[pl.ds(t * 16, 16)]
            mg = plsc.load_gather(mloc, [didx])
            e_v[pl.ds(t * 16, 16)] = jnp.exp(a_v[pl.ds(t * 16, 16)] - mg)
        pltpu.sync_copy(e_v, e_hbm.at[pl.ds(base, EBF)])
        pltpu.sync_copy(e_v, sshare.at[dst_v], add=True)
        return carry

    lax.fori_loop(0, _RF, row, 0)
    plsc.subcore_barrier()

    @pl.when(sid == 0)
    def _():
        pltpu.sync_copy(sshare, buf)
        pltpu.sync_copy(buf, spart_hbm.at[pl.ds(cid * NP, NP)])


def _edge_exp_sums(araw, dst, mpart):
    kern = pl.kernel(
        _k4_body,
        out_type=(jax.ShapeDtypeStruct((EP,), jnp.float32),
                  jax.ShapeDtypeStruct((_NC * NP,), jnp.float32)),
        mesh=_MESH,
        compiler_params=_SCPARAMS,
        scratch_types=[
            pltpu.VMEM((EBF,), jnp.float32),
            pltpu.VMEM((EBF,), jnp.int32),
            pltpu.VMEM((EBF,), jnp.float32),
            pltpu.VMEM((NP,), jnp.float32),
            pltpu.VMEM((NP,), jnp.float32),
            pltpu.VMEM_SHARED((NP,), jnp.float32),
            pltpu.SemaphoreType.DMA,
        ],
    )
    return kern(araw, dst, mpart)


def _k5_body(e_hbm, dst_hbm, spart_hbm, stab_hbm, cvec_hbm, out_hbm,
             e_vA, dst_vA, srowsA, e_vB, dst_vB, srowsB, out_v,
             sloc, buf, cvec_v, semA, semB):
    w = _worker_id()
    pltpu.sync_copy(spart_hbm.at[pl.ds(0, NP)], sloc)
    pltpu.sync_copy(spart_hbm.at[pl.ds(NP, NP)], buf)

    def ad(t, carry):
        sloc[pl.ds(t * 16, 16)] = (sloc[pl.ds(t * 16, 16)]
                                   + buf[pl.ds(t * 16, 16)])
        return carry
    lax.fori_loop(0, NP // 16, ad, 0)
    pltpu.sync_copy(cvec_hbm, cvec_v)
    pwv = cvec_v[3, pl.ds(0, 16)]
    mpv = cvec_v[4, pl.ds(0, 16)]
    mppv = cvec_v[5, pl.ds(0, 16)]
    sumwv = cvec_v[6, pl.ds(0, 16)]
    c0v = cvec_v[7, pl.ds(0, 16)]

    def fire(i, e_v, dst_v, srows, sem):
        base = (w * _RF + i) * EBF
        pltpu.sync_copy(e_hbm.at[pl.ds(base, EBF)], e_v)
        pltpu.sync_copy(dst_hbm.at[pl.ds(base, EBF)], dst_v)
        pltpu.async_copy(stab_hbm.at[dst_v], srows, sem)

    def drain(srows, sem):
        pltpu.make_async_copy(stab_hbm.at[pl.ds(0, EBF)], srows, sem).wait()

    def compute(i, e_v, dst_v, srows):
        base = (w * _RF + i) * EBF
        for t in range(EBF // 16):
            didx = dst_v[pl.ds(t * 16, 16)]
            sg = plsc.load_gather(sloc, [didx])
            a = e_v[pl.ds(t * 16, 16)] / (sg + 1e-16)
            ridx = t * 16 + lax.iota(jnp.int32, 16)

            cols = [plsc.load_gather(
                        srows, [ridx, jnp.full((16,), j, jnp.int32)])
                    for j in range(18)]
            (vx, vr, cxr, uw, xgw, rgw, mu, mxg, mrg, muu, mup,
             m_uxg, m_urg, m_pxg, m_prg, m_xg2, m_xgrg, m_rg2) = cols
            a2 = a * a
            s1sq = vx + 2.0 * a * cxr + a2 * vr + 1e-5
            rs1 = _rsqrt16(s1sq)
            zw = a * uw + pwv + (xgw + a * rgw) * rs1
            muz = a * mu + mpv + (mxg + a * mrg) * rs1
            m_a2 = a2 * muu + 2.0 * a * mup + mppv
            m_ab = a * m_uxg + a2 * m_urg + m_pxg + a * m_prg
            m_b2 = m_xg2 + 2.0 * a * m_xgrg + a2 * m_rg2
            varz = m_a2 + 2.0 * m_ab * rs1 + m_b2 / s1sq - muz * muz
            out_v[pl.ds(t * 16, 16)] = ((zw - muz * sumwv)
                                        * _rsqrt16(varz + 1e-5) + c0v)
        pltpu.sync_copy(out_v, out_hbm.at[pl.ds(base, EBF)])

    fire(0, e_vA, dst_vA, srowsA, semA)

    def pair(i, carry):
        g = i * 2
        fire(g + 1, e_vB, dst_vB, srowsB, semB)
        drain(srowsA, semA)
        compute(g, e_vA, dst_vA, srowsA)

        @pl.when(g + 2 < _RF)
        def _():
            fire(g + 2, e_vA, dst_vA, srowsA, semA)
        drain(srowsB, semB)
        compute(g + 1, e_vB, dst_vB, srowsB)
        return carry

    lax.fori_loop(0, _RF // 2, pair, 0)


def _edge_final(ev, dst, spart, stab, cvec):
    kern = pl.kernel(
        _k5_body,
        out_type=jax.ShapeDtypeStruct((EP,), jnp.float32),
        mesh=_MESH,
        compiler_params=_SCPARAMS,
        scratch_types=[
            pltpu.VMEM((EBF,), jnp.float32),
            pltpu.VMEM((EBF,), jnp.int32),
            pltpu.VMEM((EBF, 32), jnp.float32),
            pltpu.VMEM((EBF,), jnp.float32),
            pltpu.VMEM((EBF,), jnp.int32),
            pltpu.VMEM((EBF, 32), jnp.float32),
            pltpu.VMEM((EBF,), jnp.float32),
            pltpu.VMEM((NP,), jnp.float32),
            pltpu.VMEM((NP,), jnp.float32),
            pltpu.VMEM((8, D), jnp.float32),
            pltpu.SemaphoreType.DMA,
            pltpu.SemaphoreType.DMA,
        ],
    )
    return kern(ev, dst, spart, stab, cvec)


def kernel(edge_index, x, Wq, bq, Wk, bk, Wv, bv, ln_g, ln_b,
           W1, b1, W2, b2, W3, b3, Wvec, bvec, fn_g, fn_b):
    ei = edge_index.astype(jnp.int32)
    ne = ei.shape[1]
    src = jnp.pad(ei[0], (0, EP - ne))
    dst = jnp.pad(ei[1], (0, EP - ne), constant_values=NP - 1)
    L = Wq.shape[0] - 1

    brow = jnp.zeros((8, 1024), jnp.float32)
    brow = brow.at[0, :].set(b1)
    brow = brow.at[1, :512].set(b2)
    brow = brow.at[2, :D].set(b3)
    brow = brow.at[3, :D].set(ln_b[L])
    brow = brow.at[4, :D].set(fn_g)
    brow = brow.at[5, :D].set(fn_b)
    brow = brow.at[6, :D].set(Wvec[0])
    brow = brow.at[7, :].set(bvec[0])
    wct, cvec = _fold_weights(W1.T, W2.T, W3.T, brow)

    xpad = jnp.pad(x, ((0, NP - x.shape[0]), (0, 0)))
    vrows = jnp.zeros((8, D), jnp.float32)
    vrows = vrows.at[0].set(bq[L]).at[1].set(bk[L]).at[2].set(bv[L])
    vrows = vrows.at[3].set(ln_g[L])
    q, k, s128 = _node_precompute(xpad, Wq[L].T, Wk[L].T, Wv[L].T, wct,
                                  vrows, cvec)

    araw, mpart, s32 = _edge_dots(src, dst, q, k, s128)
    ev, spart = _edge_exp_sums(araw, dst, mpart)
    return _edge_final(ev, dst, spart, s32, cvec)[:ne]
